# baseline (speedup 1.0000x reference)
import jax
import jax.numpy as jnp
from jax import lax
from jax.experimental import pallas as pl
from jax.experimental.pallas import tpu as pltpu
from jax.experimental.pallas import tpu_sc as plsc

EMBED = 128
HIDDEN = 1024
BATCH = 16384
NC = 2
NS = 16
NW = NC * NS

SIZES = (2048, 2048, 4096, 8192)
MAX_IDX = 128
BM = 2048


def _make_gather(offset, size):
    bpw = size // NW
    nchunk = -(-bpw // MAX_IDX)
    sub = bpw // nchunk
    assert sub * nchunk == bpw and sub % 8 == 0

    def body(xp, xo, xs, wp, wo, ws, op, oo, osub,
             ip, io, isub, rp, ro, rs, gsem, wsem):
        wid = lax.axis_index("s") * NC + lax.axis_index("c")
        obase = wid * bpw
        ibase = offset + obase
        tabs = ((xp, wp, op, ip, rp), (xo, wo, oo, io, ro),
                (xs, ws, osub, isub, rs))
        for x_hbm, _, _, iv, _ in tabs:
            pltpu.sync_copy(x_hbm.at[pl.ds(ibase, bpw)], iv)
        for _, t_hbm, _, iv, rv in tabs:
            for j in range(nchunk):
                pltpu.async_copy(
                    t_hbm.at[iv.at[pl.ds(j * sub, sub)]],
                    rv.at[pl.ds(j * sub, sub)], gsem)
        for _, t_hbm, o_hbm, iv, rv in tabs:
            for j in range(nchunk):
                pltpu.make_async_copy(
                    t_hbm.at[iv.at[pl.ds(j * sub, sub)]],
                    rv.at[pl.ds(j * sub, sub)], gsem).wait()
            pltpu.async_copy(rv, o_hbm.at[pl.ds(obase, bpw)], wsem)
        for _, _, o_hbm, _, rv in tabs:
            pltpu.make_async_copy(
                rv, o_hbm.at[pl.ds(obase, bpw)], wsem).wait()

    h_type = jax.ShapeDtypeStruct((size, EMBED), jnp.float32)
    return pl.kernel(
        body,
        mesh=plsc.VectorSubcoreMesh(core_axis_name="c", subcore_axis_name="s"),
        out_type=(h_type, h_type, h_type),
        scratch_types=[
            pltpu.VMEM((bpw,), jnp.int32),
            pltpu.VMEM((bpw,), jnp.int32),
            pltpu.VMEM((bpw,), jnp.int32),
            pltpu.VMEM((bpw, EMBED), jnp.float32),
            pltpu.VMEM((bpw, EMBED), jnp.float32),
            pltpu.VMEM((bpw, EMBED), jnp.float32),
            pltpu.SemaphoreType.DMA,
            pltpu.SemaphoreType.DMA,
        ],
    )


_gathers = [_make_gather(off, size)
            for off, size in zip((0, 2048, 4096, 8192), SIZES)]


def _mlp_first_body(hp, ho, hs, w, b, o):
    h = jnp.concatenate(
        (
            jnp.maximum(hp[...], 0.0),
            jnp.maximum(ho[...], 0.0),
            jnp.maximum(hs[...], 0.0),
        ),
        axis=1,
    ).astype(jnp.bfloat16)
    acc = jnp.dot(h, w[...], preferred_element_type=jnp.float32)
    o[...] = jnp.maximum(acc + b[...], 0.0)


def _mlp_next_body(hp, ho, hs, w, b, prev, o):
    del prev
    _mlp_first_body(hp, ho, hs, w, b, o)


_OUT_TYPE = jax.ShapeDtypeStruct((BATCH, HIDDEN), jnp.float32)

_H_SPECS = [
    pl.BlockSpec((BM, EMBED), lambda i: (i, 0)),
    pl.BlockSpec((BM, EMBED), lambda i: (i, 0)),
    pl.BlockSpec((BM, EMBED), lambda i: (i, 0)),
    pl.BlockSpec((3 * EMBED, HIDDEN), lambda i: (0, 0)),
    pl.BlockSpec((1, HIDDEN), lambda i: (0, 0)),
]


def _mlp_chunk(offset, size, hp, ho, hs, w, b, prev=None):
    off = offset // BM
    out_spec = pl.BlockSpec((BM, HIDDEN), lambda i: (i + off, 0))
    if prev is None:
        return pl.pallas_call(
            _mlp_first_body,
            grid=(size // BM,),
            in_specs=_H_SPECS,
            out_specs=out_spec,
            out_shape=_OUT_TYPE,
        )(hp, ho, hs, w, b)
    return pl.pallas_call(
        _mlp_next_body,
        grid=(size // BM,),
        in_specs=_H_SPECS + [pl.BlockSpec(memory_space=pl.ANY)],
        out_specs=out_spec,
        out_shape=_OUT_TYPE,
        input_output_aliases={5: 0},
    )(hp, ho, hs, w, b, prev)


def kernel(X_phase, X_occurrence, X_subject, X_lengths,
           W_phase, W_occurrence, W_subject, W_out, b_out):
    del X_lengths
    xp = X_phase.astype(jnp.int32)
    xo = X_occurrence.astype(jnp.int32)
    xs = X_subject.astype(jnp.int32)
    w_bf = W_out.astype(jnp.bfloat16)
    b2d = b_out.reshape(1, HIDDEN)

    h_chunks = [g(xp, xo, xs, W_phase, W_occurrence, W_subject)
                for g in _gathers]
    out = None
    offset = 0
    for size, (hp, ho, hs) in zip(SIZES, h_chunks):
        out = _mlp_chunk(offset, size, hp, ho, hs, w_bf, b2d, out)
        offset += size
    return out


# speedup vs baseline: 1.1394x; 1.1394x over previous
import jax
import jax.numpy as jnp
from jax import lax
from jax.experimental import pallas as pl
from jax.experimental.pallas import tpu as pltpu
from jax.experimental.pallas import tpu_sc as plsc

EMBED = 128
HIDDEN = 1024
BATCH = 16384
NC = 2
NS = 16
NW = NC * NS
B_PER_W = BATCH // NW
UNIT = 128
UNITS_PER_TAB = B_PER_W // UNIT
NUNITS = 3 * UNITS_PER_TAB
NBUF = 4


def _gather_body(xp, xo, xs, wp, wo, ws, op, oo, osub,
                 i0, i1, i2, b0, b1, b2, b3, gsem, wsem):
    wid = lax.axis_index("s") * NC + lax.axis_index("c")
    base = wid * B_PER_W
    idxs = (i0, i1, i2)
    tabs = (wp, wo, ws)
    outs = (op, oo, osub)
    bufs = (b0, b1, b2, b3)
    for x_hbm, iv in zip((xp, xo, xs), idxs):
        pltpu.sync_copy(x_hbm.at[pl.ds(base, B_PER_W)], iv)

    def g(u):
        t, j = divmod(u, UNITS_PER_TAB)
        return (tabs[t].at[idxs[t].at[pl.ds(j * UNIT, UNIT)]], bufs[u % NBUF])

    def w(u):
        t, j = divmod(u, UNITS_PER_TAB)
        return (bufs[u % NBUF], outs[t].at[pl.ds(base + j * UNIT, UNIT)])

    for u in range(NBUF):
        pltpu.async_copy(*g(u), gsem)
    for u in range(NUNITS):
        pltpu.make_async_copy(*g(u), gsem).wait()
        pltpu.async_copy(*w(u), wsem)
        if u + NBUF < NUNITS:
            pltpu.make_async_copy(*w(u), wsem).wait()
            pltpu.async_copy(*g(u + NBUF), gsem)
    for u in range(NUNITS - NBUF, NUNITS):
        pltpu.make_async_copy(*w(u), wsem).wait()


_h_type = jax.ShapeDtypeStruct((BATCH, EMBED), jnp.float32)

_gather = pl.kernel(
    _gather_body,
    mesh=plsc.VectorSubcoreMesh(core_axis_name="c", subcore_axis_name="s"),
    out_type=(_h_type, _h_type, _h_type),
    scratch_types=(
        [pltpu.VMEM((B_PER_W,), jnp.int32)] * 3
        + [pltpu.VMEM((UNIT, EMBED), jnp.float32)] * NBUF
        + [pltpu.SemaphoreType.DMA, pltpu.SemaphoreType.DMA]
    ),
)


BM = 4096


def _mlp_body(hp, ho, hs, w, b, o):
    h = jnp.concatenate(
        (
            jnp.maximum(hp[...], 0.0),
            jnp.maximum(ho[...], 0.0),
            jnp.maximum(hs[...], 0.0),
        ),
        axis=1,
    ).astype(jnp.bfloat16)
    acc = jnp.dot(h, w[...], preferred_element_type=jnp.float32)
    o[...] = jnp.maximum(acc + b[...], 0.0)


def _mlp(hp, ho, hs, w, b):
    return pl.pallas_call(
        _mlp_body,
        grid=(BATCH // BM,),
        in_specs=[
            pl.BlockSpec((BM, EMBED), lambda i: (i, 0)),
            pl.BlockSpec((BM, EMBED), lambda i: (i, 0)),
            pl.BlockSpec((BM, EMBED), lambda i: (i, 0)),
            pl.BlockSpec((3 * EMBED, HIDDEN), lambda i: (0, 0)),
            pl.BlockSpec((1, HIDDEN), lambda i: (0, 0)),
        ],
        out_specs=pl.BlockSpec((BM, HIDDEN), lambda i: (i, 0)),
        out_shape=jax.ShapeDtypeStruct((BATCH, HIDDEN), jnp.float32),
    )(hp, ho, hs, w, b)


def kernel(X_phase, X_occurrence, X_subject, X_lengths,
           W_phase, W_occurrence, W_subject, W_out, b_out):
    del X_lengths
    hp, ho, hs = _gather(
        X_phase.astype(jnp.int32),
        X_occurrence.astype(jnp.int32),
        X_subject.astype(jnp.int32),
        W_phase, W_occurrence, W_subject,
    )
    return _mlp(hp, ho, hs, W_out.astype(jnp.bfloat16), b_out.reshape(1, HIDDEN))
